# Initial kernel scaffold; baseline (speedup 1.0000x reference)
import jax
import jax.numpy as jnp
from jax.experimental import pallas as pl


def kernel(node_feature, edge_index, edge_feature, line_edge_index, node_edge_index, edge_node_index, node_edge_scatter_index, edge_node_scatter_index, params):
    raise NotImplementedError("write your pallas kernel here")


# speedup vs baseline: 1.0967x; 1.0967x over previous
import functools

import jax
import jax.numpy as jnp
from jax.experimental import pallas as pl
from jax.experimental.pallas import tpu as pltpu

N = 10000
E = 320000
H = 128


def _combine_body(x1, x2, x3, w1, w2, w3, b, o, *, act, norm):
    out = jnp.dot(x1[...], w1[...], preferred_element_type=jnp.float32)
    out += jnp.dot(x2[...], w2[...], preferred_element_type=jnp.float32)
    out += jnp.dot(x3[...], w3[...], preferred_element_type=jnp.float32)
    out += b[...]
    if norm:
        nrm = jnp.sqrt(jnp.sum(out * out, axis=1, keepdims=True))
        out = out / jnp.maximum(nrm, 1e-12)
    if act:
        out = jnp.where(out >= 0, out, 0.01 * out)
    o[...] = out


def _combine(x1, x2, x3, w1, w2, w3, b, act, norm, block_rows):
    rows = x1.shape[0]
    grid = pl.cdiv(rows, block_rows)
    kin = [x.shape[1] for x in (x1, x2, x3)]
    return pl.pallas_call(
        functools.partial(_combine_body, act=act, norm=norm),
        grid=(grid,),
        in_specs=[
            pl.BlockSpec((block_rows, kin[0]), lambda i: (i, 0)),
            pl.BlockSpec((block_rows, kin[1]), lambda i: (i, 0)),
            pl.BlockSpec((block_rows, kin[2]), lambda i: (i, 0)),
            pl.BlockSpec((kin[0], H), lambda i: (0, 0)),
            pl.BlockSpec((kin[1], H), lambda i: (0, 0)),
            pl.BlockSpec((kin[2], H), lambda i: (0, 0)),
            pl.BlockSpec((1, H), lambda i: (0, 0)),
        ],
        out_specs=pl.BlockSpec((block_rows, H), lambda i: (i, 0)),
        out_shape=jax.ShapeDtypeStruct((rows, H), jnp.float32),
    )(x1, x2, x3, w1, w2, w3, b)


def _final_body(x, w, b, o, mx):
    out = jnp.dot(x[...], w[...], preferred_element_type=jnp.float32) + b[...]
    o[...] = out
    @pl.when(pl.program_id(0) == 0)
    def _():
        mx[...] = jnp.full_like(mx, -jnp.inf)
    mx[...] = jnp.maximum(mx[...], jnp.max(out, axis=0, keepdims=True))


def _final(x, w, b, block_rows):
    rows = x.shape[0]
    grid = pl.cdiv(rows, block_rows)
    return pl.pallas_call(
        _final_body,
        grid=(grid,),
        in_specs=[
            pl.BlockSpec((block_rows, H), lambda i: (i, 0)),
            pl.BlockSpec((H, H), lambda i: (0, 0)),
            pl.BlockSpec((1, H), lambda i: (0, 0)),
        ],
        out_specs=[
            pl.BlockSpec((block_rows, H), lambda i: (i, 0)),
            pl.BlockSpec((1, H), lambda i: (0, 0)),
        ],
        out_shape=[
            jax.ShapeDtypeStruct((rows, H), jnp.float32),
            jax.ShapeDtypeStruct((1, H), jnp.float32),
        ],
    )(x, w, b)


def _gsum(src, gidx, sidx, nseg):
    return jax.ops.segment_sum(src[gidx], sidx, num_segments=nseg)


def kernel(node_feature, edge_index, edge_feature, line_edge_index,
           node_edge_index, edge_node_index, node_edge_scatter_index,
           edge_node_scatter_index, params):
    nf, ef = node_feature, edge_feature
    row, col = edge_index[0], edge_index[1]
    lrow, lcol = line_edge_index[0], line_edge_index[1]

    for i in range(2):
        pn_, pe_ = params["node"][i], params["edge"][i]
        act = (i != 1)
        A = _gsum(nf, row, col, N)
        B = _gsum(ef, node_edge_index, node_edge_scatter_index, N)
        C = _gsum(ef, lrow, lcol, E)
        D = _gsum(nf, edge_node_index, edge_node_scatter_index, E)
        nf_new = _combine(
            nf, A, B,
            pn_["center"]["W"].T, pn_["neigh"]["W"].T, pn_["edge"]["W"].T,
            (pn_["center"]["b"] + pn_["neigh"]["b"] + pn_["edge"]["b"])[None, :],
            act, True, 1000)
        ef_new = _combine(
            ef, C, D,
            pe_["center"]["W"].T, pe_["neigh"]["W"].T, pe_["edge"]["W"].T,
            (pe_["center"]["b"] + pe_["neigh"]["b"] + pe_["edge"]["b"])[None, :],
            act, True, 2000)
        nf, ef = nf_new, ef_new

    tn, pn = _final(nf, params["node_lin"]["W"].T, params["node_lin"]["b"][None, :], 1000)
    te, pe = _final(ef, params["edge_lin"]["W"].T, params["edge_lin"]["b"][None, :], 2000)
    return (pn + pe, tn, te)
